# baseline (speedup 1.0000x reference)
import numpy as np
import jax
import jax.numpy as jnp
from jax.experimental import pallas as pl
from jax.experimental.pallas import tpu as pltpu

H = 12
S = 2048
D = 64
NH = 12
K = 624
RB = 128
C132 = float(np.float32(-1.0) / np.float32(132.0))


def _lane_phase(acc):
    y = acc.reshape(acc.shape[:-1] + (16, 8))
    p8 = y[..., 0, :]
    for v in range(1, 16):
        p8 = p8 + y[..., v, :]
    s4 = p8[..., :4] + p8[..., 4:]
    s2 = s4[..., :2] + s4[..., 2:]
    return s2[..., 0] + s2[..., 1]


def _scores_kernel(x_ref, o_ref):
    hg = pl.program_id(1)
    x = x_ref[...]
    x = x.reshape(4, RB, S)
    p = jnp.clip(x, 1e-10, 1.0)

    accz = p[:, :, 0:128]
    for c in range(1, 16):
        accz = accz + p[:, :, c * 128:(c + 1) * 128]
    z = _lane_phase(accz)

    q = p / z[:, :, None]
    t = q * jnp.log2(q)

    acc = t[0, :, 0:128]
    first = True
    for c in range(16):
        for dh in range(4):
            if first:
                first = False
                continue
            acc = acc + t[dh, :, c * 128:(c + 1) * 128]
    s = _lane_phase(acc)
    s3 = s.reshape(1, 1, RB)

    @pl.when(hg == 0)
    def _():
        o_ref[...] = s3

    @pl.when(hg == 1)
    def _():
        o_ref[...] = o_ref[...] + s3

    @pl.when(hg == 2)
    def _():
        o_ref[...] = (o_ref[...] + s3) * C132


def _select_kernel(s_ref, o_ref):
    b = jax.lax.bitcast_convert_type(s_ref[...], jnp.int32)

    def search_body(_, carry):
        lo, hi = carry
        mid = lo + ((hi - lo + 1) >> 1)
        cnt = jnp.sum((b >= mid).astype(jnp.int32))
        big = cnt >= K
        return (jnp.where(big, mid, lo), jnp.where(big, hi, mid - 1))

    lo, hi = jax.lax.fori_loop(0, 31, search_body,
                               (jnp.int32(0), jnp.int32(0x48000000)))
    vstar = lo
    gt = b > vstar
    eq = b == vstar
    n_gt = jnp.sum(gt.astype(jnp.int32))

    def rowmajor_cumsum(m):
        lane = m
        for sh in (1, 2, 4, 8, 16, 32, 64):
            z = jnp.zeros((16, sh), jnp.int32)
            lane = lane + jnp.concatenate([z, lane[:, :128 - sh]], axis=1)
        rows = jnp.sum(m, axis=1, keepdims=True)
        rp = rows
        for sh in (1, 2, 4, 8):
            z = jnp.zeros((sh, 1), jnp.int32)
            rp = rp + jnp.concatenate([z, rp[:16 - sh, :]], axis=0)
        return lane + (rp - rows)

    eq_rank = rowmajor_cumsum(eq.astype(jnp.int32))
    keep = gt | (eq & (eq_rank <= (K - n_gt)))
    csum = rowmajor_cumsum(keep.astype(jnp.int32))

    cflat = csum.reshape(1, 1, S)
    ja = jax.lax.broadcasted_iota(jnp.int32, (5, 128, 1), 0)
    jb = jax.lax.broadcasted_iota(jnp.int32, (5, 128, 1), 1)
    jidx = ja * 128 + jb
    m = (cflat < (jidx + 1)).astype(jnp.int32)
    o_ref[...] = jnp.sum(m, axis=-1)


def _gather_kernel(idx_ref, k_ref, v_ref, ok_ref, ov_ref):
    def body(j, _):
        i = idx_ref[j]
        ok_ref[pl.ds(j, 1), :] = k_ref[pl.ds(i, 1), :]
        ov_ref[pl.ds(j, 1), :] = v_ref[pl.ds(i, 1), :]
        return 0

    jax.lax.fori_loop(0, K, body, 0)


def kernel(attn, key_cache, value_cache):
    scores = pl.pallas_call(
        _scores_kernel,
        grid=(S // RB, 3),
        in_specs=[pl.BlockSpec((4, 1, RB, S), lambda rb, hg: (hg, 0, rb, 0))],
        out_specs=pl.BlockSpec((1, 1, RB), lambda rb, hg: (rb, 0, 0)),
        out_shape=jax.ShapeDtypeStruct((S // RB, 1, RB), jnp.float32),
    )(attn)
    scores = scores.reshape(16, 128)

    idx640 = pl.pallas_call(
        _select_kernel,
        in_specs=[pl.BlockSpec((16, 128), lambda: (0, 0))],
        out_specs=pl.BlockSpec((5, 128), lambda: (0, 0)),
        out_shape=jax.ShapeDtypeStruct((5, 128), jnp.int32),
    )(scores)
    idx = idx640.reshape(640)[:K]

    k2 = key_cache.reshape(S, NH * D)
    v2 = value_cache.reshape(S, NH * D)
    ck2, cv2 = pl.pallas_call(
        _gather_kernel,
        in_specs=[
            pl.BlockSpec(memory_space=pltpu.SMEM),
            pl.BlockSpec((S, NH * D), lambda: (0, 0)),
            pl.BlockSpec((S, NH * D), lambda: (0, 0)),
        ],
        out_specs=[
            pl.BlockSpec((K, NH * D), lambda: (0, 0)),
            pl.BlockSpec((K, NH * D), lambda: (0, 0)),
        ],
        out_shape=[
            jax.ShapeDtypeStruct((K, NH * D), jnp.float32),
            jax.ShapeDtypeStruct((K, NH * D), jnp.float32),
        ],
    )(idx, k2, v2)

    ck = ck2.reshape(1, K, NH, D)
    cv = cv2.reshape(1, K, NH, D)
    return ck, cv, idx.reshape(1, K)
